# Initial kernel scaffold; baseline (speedup 1.0000x reference)
import jax
import jax.numpy as jnp
from jax.experimental import pallas as pl


def kernel(x, adj, W1, a_src1, a_dst1, b1, W2, a_src2, a_dst2, b2):
    raise NotImplementedError("write your pallas kernel here")


# speedup vs baseline: 4.5557x; 4.5557x over previous
import functools

import jax
import jax.numpy as jnp
from jax import lax
from jax.experimental import pallas as pl
from jax.experimental.pallas import tpu as pltpu
from jax.experimental.pallas import tpu_sc as plsc

N_NODES = 10000
CHUNK = 16


def _mm_body(x_ref, w_ref, o_ref):
    o_ref[...] = jnp.dot(x_ref[...], w_ref[...],
                         preferred_element_type=jnp.float32)


def _matmul(x, w, tile_m=400):
    m, k = x.shape
    n = w.shape[1]
    grid = m // tile_m
    return pl.pallas_call(
        _mm_body,
        grid=(grid,),
        in_specs=[
            pl.BlockSpec((tile_m, k), lambda i: (i, 0)),
            pl.BlockSpec((k, n), lambda i: (0, 0)),
        ],
        out_specs=pl.BlockSpec((tile_m, n), lambda i: (i, 0)),
        out_shape=jax.ShapeDtypeStruct((m, n), jnp.float32),
    )(x, w)


def _sc_gat_layer(y, adp, srcs, dsts, bounds, bias, *, heads, feat,
                  n_pass, npt, apply_elu):
    fh = feat * heads
    row_w = fh + 128
    assert y.shape[1] == row_w
    assert npt % 8 == 0
    ncg = row_w // 16
    mesh = plsc.VectorSubcoreMesh(core_axis_name="c", subcore_axis_name="s")

    @functools.partial(
        pl.kernel,
        out_type=jax.ShapeDtypeStruct((N_NODES, fh), jnp.float32),
        mesh=mesh,
        scratch_types=[
            pltpu.VMEM((n_pass * 32, 16), jnp.int32),
            pltpu.VMEM((CHUNK,), jnp.int32),
            pltpu.VMEM((CHUNK,), jnp.int32),
            pltpu.VMEM((2 * CHUNK,), jnp.int32),
            pltpu.VMEM((CHUNK, row_w), jnp.float32),
            pltpu.VMEM((CHUNK, 128), jnp.float32),
            pltpu.VMEM((npt, row_w), jnp.float32),
            pltpu.VMEM((8, fh), jnp.float32),
            pltpu.VMEM((fh,), jnp.float32),
        ],
    )
    def k(y_hbm, adp_hbm, srcs_hbm, dsts_hbm, bounds_hbm, bias_hbm, out_hbm,
          bounds_v, si_v, di_v, dl_v, rows_v, adg_v, acc, or_v, b_v):
        c = lax.axis_index("c")
        s = lax.axis_index("s")
        w = c * 16 + s
        pltpu.sync_copy(bounds_hbm, bounds_v)
        pltpu.sync_copy(bias_hbm, b_v)
        zv = jnp.zeros((16,), jnp.float32)

        def edge_chunk(chunk_idx, e_lo, e_hi, node_lo):
            base = chunk_idx * CHUNK
            pltpu.sync_copy(srcs_hbm.at[pl.ds(base, CHUNK)], si_v)
            pltpu.sync_copy(dsts_hbm.at[pl.ds(base, CHUNK)], di_v)
            dl_v[pl.ds(0, 16)] = jnp.clip(di_v[...] - node_lo, 0, npt - 1)
            pltpu.sync_copy(y_hbm.at[si_v], rows_v)
            pltpu.sync_copy(adp_hbm.at[di_v], adg_v)

            def per_edge(e, _):
                av = rows_v[e, pl.ds(fh, 16)]
                dv = adg_v[e, pl.ds(0, 16)]
                v = av + dv
                v = jnp.where(v >= 0, v, 0.2 * v)
                ge = base + e
                mf = jnp.where((ge >= e_lo) & (ge < e_hi),
                               jnp.float32(1.0), jnp.float32(0.0))
                sv = jnp.exp(v) * jnp.full((16,), mf)
                n = dl_v[pl.ds(e, 16)][0]
                acc[n, pl.ds(fh, 16)] = acc[n, pl.ds(fh, 16)] + sv
                for h in range(heads):
                    shv = jnp.full((16,), sv[h])
                    for j in range(feat // 16):
                        col = h * feat + j * 16
                        acc[n, pl.ds(col, 16)] = acc[n, pl.ds(col, 16)] \
                            + rows_v[e, pl.ds(col, 16)] * shv
                return 0

            lax.fori_loop(0, CHUNK, per_edge, 0)
            return 0

        def pass_body(p, _):
            row = p * 32 + w
            bv = bounds_v[row, pl.ds(0, 16)]
            ch_lo, ch_hi, e_lo, e_hi = bv[0], bv[1], bv[2], bv[3]
            node_lo = row * npt

            def zero_body(r, _):
                for j in range(ncg):
                    acc[r, pl.ds(j * 16, 16)] = zv
                return 0

            lax.fori_loop(0, npt, zero_body, 0)

            def chunk_body(ki, _):
                edge_chunk(ch_lo + ki, e_lo, e_hi, node_lo)
                return 0

            lax.fori_loop(0, ch_hi - ch_lo, chunk_body, 0)

            def div_body(g, _):
                ng = node_lo + g * 8

                @pl.when(ng < N_NODES)
                def _():
                    def div_row(r, _2):
                        den = acc[g * 8 + r, pl.ds(fh, 16)]
                        for h in range(heads):
                            dhv = jnp.full((16,), den[h] + 1e-16)
                            for j in range(feat // 16):
                                col = h * feat + j * 16
                                o = acc[g * 8 + r, pl.ds(col, 16)] / dhv \
                                    + b_v[pl.ds(col, 16)]
                                if apply_elu:
                                    o = jnp.where(o > 0, o, jnp.exp(
                                        jnp.minimum(o, 0.0)) - 1.0)
                                or_v[r, pl.ds(col, 16)] = o
                        return 0

                    lax.fori_loop(0, 8, div_row, 0)
                    pltpu.sync_copy(or_v, out_hbm.at[pl.ds(ng, 8)])

                return 0

            lax.fori_loop(0, npt // 8, div_body, 0)
            return 0

        lax.fori_loop(0, n_pass, pass_body, 0)

    return k(y, adp, srcs, dsts, bounds, bias)


def _fold_attn(W, a_src, a_dst, heads, feat):
    k = W.shape[0]
    As = jnp.einsum("khf,hf->kh", W.reshape(k, heads, feat), a_src)
    Ad = jnp.einsum("khf,hf->kh", W.reshape(k, heads, feat), a_dst)
    pad = 16 - heads
    As = jnp.pad(As, ((0, 0), (0, pad)))
    Ad = jnp.pad(Ad, ((0, 0), (0, pad)))
    Z = jnp.zeros((k, 96), jnp.float32)
    return jnp.concatenate([W, As, Ad, Z], axis=1)


def _bounds(dsts, n_pass, npt):
    los = jnp.arange(n_pass * 32, dtype=jnp.int32) * npt
    his = jnp.minimum(los + npt, N_NODES)
    e_lo = jnp.searchsorted(dsts, los).astype(jnp.int32)
    e_hi = jnp.searchsorted(dsts, his).astype(jnp.int32)
    ch_lo = e_lo // CHUNK
    ch_hi = (e_hi + CHUNK - 1) // CHUNK
    b = jnp.stack([ch_lo, ch_hi, e_lo, e_hi], axis=1)
    return jnp.pad(b, ((0, 0), (0, 12)))


def kernel(x, adj, W1, a_src1, a_dst1, b1, W2, a_src2, a_dst2, b2):
    order = jnp.argsort(adj[1])
    srcs = adj[0][order].astype(jnp.int32)
    dsts = adj[1][order].astype(jnp.int32)

    y1 = _matmul(x, _fold_attn(W1, a_src1, a_dst1, 8, 128))
    adp1 = jnp.pad(y1[:, 1040:1056], ((0, 0), (0, 112)))
    h2 = _sc_gat_layer(y1, adp1, srcs, dsts, _bounds(dsts, 5, 64), b1,
                       heads=8, feat=128, n_pass=5, npt=64, apply_elu=True)

    y2 = _matmul(h2, _fold_attn(W2, a_src2, a_dst2, 1, 256))
    adp2 = jnp.pad(y2[:, 272:288], ((0, 0), (0, 112)))
    out = _sc_gat_layer(y2, adp2, srcs, dsts, _bounds(dsts, 2, 160), b2,
                        heads=1, feat=256, n_pass=2, npt=160,
                        apply_elu=False)
    return out
